# baseline (speedup 1.0000x reference)
import functools

import jax
import jax.numpy as jnp
from jax import lax
from jax.experimental import pallas as pl
from jax.experimental.pallas import tpu as pltpu
from jax.experimental.pallas import tpu_sc as plsc

_B = 16384
_D = 32
_NW = 32
_BPW = _B // _NW
_CH = 4
_CB = _BPW // _CH


def _take16(v, idx):
    dnums = lax.GatherDimensionNumbers(
        offset_dims=(), collapsed_slice_dims=(0,), start_index_map=(0,))
    return lax.gather(v, idx.reshape(16, 1), dnums, (1,),
                      mode=lax.GatherScatterMode.PROMISE_IN_BOUNDS)


def _gmf_body(uidx_hbm, iidx_hbm, par_hbm, utab_hbm, itab_hbm, out_hbm,
              uidx_v, iidx_v, urows_v, irows_v, par_v, out_v, sem):
    wid = lax.axis_index("s") * 2 + lax.axis_index("c")

    pltpu.sync_copy(uidx_hbm.at[pl.ds(wid * _CH, _CH)], uidx_v)
    pltpu.sync_copy(iidx_hbm.at[pl.ds(wid * _CH, _CH)], iidx_v)
    pltpu.sync_copy(par_hbm, par_v)

    copies = []
    for k in range(_CH):
        copies.append(pltpu.async_copy(
            utab_hbm.at[uidx_v.at[k]], urows_v.at[pl.ds(k * _CB, _CB)], sem))
        copies.append(pltpu.async_copy(
            itab_hbm.at[iidx_v.at[k]], irows_v.at[pl.ds(k * _CB, _CB)], sem))
    for c in copies:
        c.wait()

    iota = lax.iota(jnp.int32, 16)
    neg_b = par_v[pl.ds(_D, 16)]
    w_lo = par_v[pl.ds(0, 16)]
    w_hi = par_v[pl.ds(16, 16)]
    fifteen = jnp.full((16,), 15, jnp.int32)

    def row_group(rg, carry):
        base = rg * 16
        acc = neg_b
        for j in range(16):
            r = base + j
            u_lo = urows_v[r, pl.ds(0, 16)]
            u_hi = urows_v[r, pl.ds(16, 16)]
            i_lo = irows_v[r, pl.ds(0, 16)]
            i_hi = irows_v[r, pl.ds(16, 16)]
            s = u_lo * i_lo * w_lo + u_hi * i_hi * w_hi
            hs = _take16(plsc.cumsum(s), fifteen)
            acc = jnp.where(iota == j, hs + neg_b, acc)
        out_v[pl.ds(base, 16)] = 1.0 / (1.0 + jnp.exp(acc))
        return carry

    lax.fori_loop(0, _BPW // 16, row_group, 0)

    pltpu.sync_copy(out_v, out_hbm.at[pl.ds(wid * _BPW, _BPW)])


def kernel(user_indices, item_indices, user_table, item_table, W, b):
    uidx = user_indices.astype(jnp.int32).reshape(_NW * _CH, _CB)
    iidx = item_indices.astype(jnp.int32).reshape(_NW * _CH, _CB)
    params = jnp.concatenate(
        [-W.reshape(_D), jnp.broadcast_to(-b, (16,))]).astype(jnp.float32)

    mesh = plsc.VectorSubcoreMesh(core_axis_name="c", subcore_axis_name="s")
    run = functools.partial(
        pl.kernel, mesh=mesh,
        compiler_params=pltpu.CompilerParams(
            needs_layout_passes=False, use_tc_tiling_on_sc=False),
        out_type=jax.ShapeDtypeStruct((_B,), jnp.float32),
        scratch_types=[
            pltpu.VMEM((_CH, _CB), jnp.int32),
            pltpu.VMEM((_CH, _CB), jnp.int32),
            pltpu.VMEM((_BPW, _D), jnp.float32),
            pltpu.VMEM((_BPW, _D), jnp.float32),
            pltpu.VMEM((_D + 16,), jnp.float32),
            pltpu.VMEM((_BPW,), jnp.float32),
            pltpu.SemaphoreType.DMA,
        ],
    )(_gmf_body)
    out = run(uidx, iidx, params, user_table, item_table)
    return out.reshape(_B, 1)
